# Initial kernel scaffold; baseline (speedup 1.0000x reference)
import jax
import jax.numpy as jnp
from jax.experimental import pallas as pl


def kernel(sentence, word_emb, pos_emb):
    raise NotImplementedError("write your pallas kernel here")


# speedup vs baseline: 3.4984x; 3.4984x over previous
import functools

import jax
import jax.numpy as jnp
from jax import lax
from jax.experimental import pallas as pl
from jax.experimental.pallas import tpu as pltpu
from jax.experimental.pallas import tpu_sc as plsc

D = 64
T = 200
B = 4096
NW = 32
BPW = B // NW
CB = 2
ROWS = CB * T
NCHUNK = BPW // CB
NIDX = CB * T // 100

_mesh = plsc.VectorSubcoreMesh(core_axis_name="c", subcore_axis_name="s")


@functools.partial(
    pl.kernel,
    out_type=jax.ShapeDtypeStruct((B * T, D), jnp.float32),
    mesh=_mesh,
    scratch_types=[
        pltpu.VMEM((NIDX, 100), jnp.int32),
        pltpu.VMEM((ROWS, D), jnp.float32),
        pltpu.VMEM((T, D), jnp.float32),
        pltpu.SemaphoreType.DMA,
    ],
    compiler_params=pltpu.CompilerParams(use_tc_tiling_on_sc=False),
)
def _emb(idx_hbm, word_hbm, pos_hbm, out_hbm, idx_v, rows_v, pos_v, sem):
    wid = lax.axis_index("s") * 2 + lax.axis_index("c")
    pltpu.sync_copy(pos_hbm, pos_v)

    def chunk_body(j, carry):
        g = wid * NCHUNK + j
        pltpu.sync_copy(idx_hbm.at[pl.ds(g * NIDX, NIDX)], idx_v)
        cps = [
            pltpu.async_copy(
                word_hbm.at[idx_v.at[i]],
                rows_v.at[pl.ds(i * 100, 100)],
                sem,
            )
            for i in range(NIDX)
        ]
        for cp in cps:
            cp.wait()

        def add_body(r, c2):
            for c in range(D // 16):
                sl = pl.ds(c * 16, 16)
                p = pos_v[r, sl]
                for h in range(CB):
                    rr = h * T + r
                    rows_v[rr, sl] = rows_v[rr, sl] + p
            return c2

        lax.fori_loop(0, T, add_body, 0)
        pltpu.sync_copy(rows_v, out_hbm.at[pl.ds(g * ROWS, ROWS)])
        return carry

    lax.fori_loop(0, NCHUNK, chunk_body, 0)


def kernel(sentence, word_emb, pos_emb):
    idx = jnp.transpose(sentence, (1, 0)).reshape(B * T // 100, 100)
    pos_slice = lax.slice(pos_emb, (1, 0), (T + 1, D))
    out = _emb(idx, word_emb, pos_slice)
    return out.reshape(B, T, D)
